# baseline (device time: 10113 ns/iter reference)
import jax
import jax.numpy as jnp
from jax import lax
from jax.experimental import pallas as pl
from jax.experimental.pallas import tpu as pltpu

BLOCK_M = 256


def kernel(x, dy, gamma):
    del gamma
    m, d = x.shape
    n_blocks = m // BLOCK_M

    def body(x_ref, dy_ref, out_ref, comm_ref, send_sem, recv_sem):
        step = pl.program_id(0)
        my_x = lax.axis_index("x")
        my_y = lax.axis_index("y")
        my_z = lax.axis_index("z")
        peer = (1 - my_x, my_y, my_z)
        barrier_sem = pltpu.get_barrier_semaphore()

        @pl.when(step == 0)
        def _():
            pl.semaphore_signal(
                barrier_sem, inc=1, device_id=peer,
                device_id_type=pl.DeviceIdType.MESH,
            )
            comm_ref[0, :, :] = jnp.zeros((2, d), jnp.float32)

        xv = x_ref[:, :]
        dyv = dy_ref[:, :]
        mu = jnp.mean(xv, axis=1, keepdims=True)
        diff = xv - mu
        var = jnp.mean(diff * diff, axis=1, keepdims=True)
        xhat = diff * lax.rsqrt(var + 1e-5)
        dgamma = jnp.sum(dyv * xhat, axis=0)
        dbeta = jnp.sum(dyv, axis=0)
        comm_ref[0, :, :] += jnp.stack([dgamma, dbeta])

        @pl.when(step == n_blocks - 1)
        def _():
            pl.semaphore_wait(barrier_sem, 1)
            rdma = pltpu.make_async_remote_copy(
                src_ref=comm_ref.at[0],
                dst_ref=comm_ref.at[1],
                send_sem=send_sem,
                recv_sem=recv_sem,
                device_id=peer,
                device_id_type=pl.DeviceIdType.MESH,
            )
            rdma.start()
            rdma.wait()
            out_ref[:, :] = comm_ref[0] + comm_ref[1]

    return pl.pallas_call(
        body,
        grid=(n_blocks,),
        out_shape=jax.ShapeDtypeStruct((2, d), jnp.float32),
        in_specs=[
            pl.BlockSpec((BLOCK_M, d), lambda i: (i, 0), memory_space=pltpu.VMEM),
            pl.BlockSpec((BLOCK_M, d), lambda i: (i, 0), memory_space=pltpu.VMEM),
        ],
        out_specs=pl.BlockSpec((2, d), lambda i: (0, 0), memory_space=pltpu.VMEM),
        scratch_shapes=[
            pltpu.VMEM((2, 2, d), jnp.float32),
            pltpu.SemaphoreType.DMA,
            pltpu.SemaphoreType.DMA,
        ],
        compiler_params=pltpu.CompilerParams(
            collective_id=0,
            dimension_semantics=("arbitrary",),
        ),
    )(x, dy)
